# Initial kernel scaffold; baseline (speedup 1.0000x reference)
import functools

import jax
import jax.numpy as jnp
from jax import lax
from jax.experimental import pallas as pl
from jax.experimental.pallas import tpu as pltpu
from jax.experimental.pallas import tpu_sc as plsc

N = 10000
E = 320000
DF = 128
DH = 16
NCLS = 7

NPADROWS = 16
NY = N + NPADROWS

NWORK = 32
GROUP = 128
GPT = 79
EPT = GPT * GROUP
EPAD = NWORK * EPT
NPAD_E = EPAD - E

DEGPAD = 10240
RPT = N // 16


def _mesh():
    return plsc.VectorSubcoreMesh(core_axis_name="c", subcore_axis_name="s")


def _sc_degree(dst3, zeros1):
    @functools.partial(
        pl.kernel,
        mesh=_mesh(),
        out_type=jax.ShapeDtypeStruct((2 * DEGPAD,), jnp.float32),
        scratch_types=[
            pltpu.VMEM((GPT, GROUP), jnp.int32),
            pltpu.VMEM((GROUP,), jnp.float32),
            pltpu.VMEM_SHARED((DEGPAD,), jnp.float32),
        ],
    )
    def k(dst_hbm, zeros_hbm, out_hbm, dst_v, ones_v, deg_sh):
        c = lax.axis_index("c")
        s = lax.axis_index("s")
        w = c * 16 + s
        for j in range(GROUP // 16):
            ones_v[pl.ds(j * 16, 16)] = jnp.ones((16,), jnp.float32)
        pltpu.sync_copy(zeros_hbm.at[pl.ds(s * 640, 640)],
                        deg_sh.at[pl.ds(s * 640, 640)])
        pltpu.sync_copy(dst_hbm.at[w], dst_v)
        plsc.subcore_barrier()

        def body(g, carry):
            pltpu.sync_copy(ones_v, deg_sh.at[dst_v.at[g]], add=True)
            return carry

        lax.fori_loop(0, GPT, body, 0)
        plsc.subcore_barrier()
        pltpu.sync_copy(deg_sh.at[pl.ds(s * 640, 640)],
                        out_hbm.at[pl.ds(c * DEGPAD + s * 640, 640)])

    return k(dst3, zeros1)


def _sc_aggregate(y, src3, dst3, zeros2):
    @functools.partial(
        pl.kernel,
        mesh=_mesh(),
        out_type=jax.ShapeDtypeStruct((2 * N, DH), jnp.float32),
        scratch_types=[
            pltpu.VMEM((GPT, GROUP), jnp.int32),
            pltpu.VMEM((GPT, GROUP), jnp.int32),
            pltpu.VMEM((GROUP, DH), jnp.float32),
            pltpu.VMEM_SHARED((N, DH), jnp.float32),
            pltpu.SemaphoreType.DMA,
        ],
    )
    def k(y_hbm, src_hbm, dst_hbm, zeros_hbm, out_hbm,
          src_v, dst_v, rows_v, acc_sh, sem):
        c = lax.axis_index("c")
        s = lax.axis_index("s")
        w = c * 16 + s
        pltpu.sync_copy(zeros_hbm.at[pl.ds(s * RPT, RPT)],
                        acc_sh.at[pl.ds(s * RPT, RPT)])
        pltpu.sync_copy(src_hbm.at[w], src_v)
        pltpu.sync_copy(dst_hbm.at[w], dst_v)
        plsc.subcore_barrier()

        def body(g, carry):
            pltpu.async_copy(y_hbm.at[src_v.at[g]], rows_v, sem).wait()
            pltpu.sync_copy(rows_v, acc_sh.at[dst_v.at[g]], add=True)
            return carry

        lax.fori_loop(0, GPT, body, 0)
        plsc.subcore_barrier()
        pltpu.sync_copy(acc_sh.at[pl.ds(s * RPT, RPT)],
                        out_hbm.at[pl.ds(c * N + s * RPT, RPT)])

    return k(y, src3, dst3, zeros2)


def _dinv(cnt_ref):
    deg = jnp.sum(cnt_ref[...], axis=1, keepdims=True) + 1.0
    row = lax.broadcasted_iota(jnp.int32, (N, 1), 0)
    deg = deg - jnp.where(row < NPAD_E, 1.0, 0.0)
    return lax.rsqrt(deg)


def _tc_pre(x, W1, cnt2):
    def body(x_ref, w_ref, cnt_ref, y_ref):
        dinv = _dinv(cnt_ref)
        xw = jnp.dot(x_ref[...], w_ref[...], preferred_element_type=jnp.float32)
        y_ref[pl.ds(0, N), :] = xw * dinv
        y_ref[pl.ds(N, NPADROWS), :] = jnp.zeros((NPADROWS, DH), jnp.float32)

    return pl.pallas_call(
        body, out_shape=jax.ShapeDtypeStruct((NY, DH), jnp.float32)
    )(x, W1, cnt2)


def _tc_mid(cnt2, y1, a0, a1, b1, W2p):
    def body(cnt_ref, y1_ref, a0_ref, a1_ref, b1_ref, w2_ref, y2_ref):
        dinv = _dinv(cnt_ref)
        out1 = (a0_ref[...] + a1_ref[...] + y1_ref[pl.ds(0, N), :]) * dinv
        h = jnp.maximum(out1 + b1_ref[...], 0.0)
        z = jnp.dot(h, w2_ref[...], preferred_element_type=jnp.float32)
        y2_ref[pl.ds(0, N), :] = z * dinv
        y2_ref[pl.ds(N, NPADROWS), :] = jnp.zeros((NPADROWS, DH), jnp.float32)

    return pl.pallas_call(
        body, out_shape=jax.ShapeDtypeStruct((NY, DH), jnp.float32)
    )(cnt2, y1, a0, a1, b1, W2p)


def _tc_post(cnt2, y2, a0, a1, b2p):
    def body(cnt_ref, y2_ref, a0_ref, a1_ref, b2_ref, o_ref):
        dinv = _dinv(cnt_ref)
        out2 = (a0_ref[...] + a1_ref[...] + y2_ref[pl.ds(0, N), :]) * dinv
        out2 = out2 + b2_ref[...]
        col = lax.broadcasted_iota(jnp.int32, (N, DH), 1)
        vals = jnp.where(col < NCLS, out2, -1e30)
        mx = jnp.max(vals, axis=1, keepdims=True)
        ex = jnp.exp(vals - mx)
        lse = mx + jnp.log(jnp.sum(ex, axis=1, keepdims=True))
        o_ref[...] = (out2 - lse)[:, 0:NCLS]

    return pl.pallas_call(
        body, out_shape=jax.ShapeDtypeStruct((N, NCLS), jnp.float32)
    )(cnt2, y2, a0, a1, b2p)


def kernel(x, edge_index, W1, b1, W2, b2):
    ei = edge_index.astype(jnp.int32)
    pad_src = (jnp.arange(NPAD_E, dtype=jnp.int32) % NPADROWS) + N
    pad_dst = jnp.arange(NPAD_E, dtype=jnp.int32) % N
    src3 = jnp.concatenate([ei[0], pad_src]).reshape(NWORK, GPT, GROUP)
    dst3 = jnp.concatenate([ei[1], pad_dst]).reshape(NWORK, GPT, GROUP)
    zeros1 = jnp.zeros((DEGPAD,), jnp.float32)
    zeros2 = jnp.zeros((N, DH), jnp.float32)
    W2p = jnp.zeros((DH, DH), jnp.float32).at[:, :NCLS].set(W2)
    b2p = jnp.zeros((DH,), jnp.float32).at[:NCLS].set(b2)

    cnt = _sc_degree(dst3, zeros1)
    cnt2 = cnt.reshape(2, DEGPAD)[:, :N].T

    y1 = _tc_pre(x, W1, cnt2)
    agg1 = _sc_aggregate(y1, src3, dst3, zeros2)
    y2 = _tc_mid(cnt2, y1, agg1[:N], agg1[N:], b1, W2p)
    agg2 = _sc_aggregate(y2, src3, dst3, zeros2)
    return _tc_post(cnt2, y2, agg2[:N], agg2[N:], b2p)


# speedup vs baseline: 34.6298x; 34.6298x over previous
import functools

import jax
import jax.numpy as jnp
from jax import lax
from jax.experimental import pallas as pl
from jax.experimental.pallas import tpu as pltpu
from jax.experimental.pallas import tpu_sc as plsc

N = 10000
E = 320000
DF = 128
DH = 16
NCLS = 7

NPADROWS = 16
NY = N + NPADROWS

NWORK = 32
GROUP = 128
GPT = 79
EPT = GPT * GROUP
EPAD = NWORK * EPT
NPAD_E = EPAD - E

DEGPAD = 10240
NACC = 10240
RPT = NACC // 16


def _mesh():
    return plsc.VectorSubcoreMesh(core_axis_name="c", subcore_axis_name="s")


def _sc_degree(dst3, zeros1):
    @functools.partial(
        pl.kernel,
        mesh=_mesh(),
        compiler_params=pltpu.CompilerParams(use_tc_tiling_on_sc=False),
        out_type=jax.ShapeDtypeStruct((2 * DEGPAD,), jnp.float32),
        scratch_types=[
            pltpu.VMEM((GPT, GROUP), jnp.int32),
            pltpu.VMEM((GROUP,), jnp.float32),
            pltpu.VMEM_SHARED((DEGPAD,), jnp.float32),
        ],
    )
    def k(dst_hbm, zeros_hbm, out_hbm, dst_v, ones_v, deg_sh):
        c = lax.axis_index("c")
        s = lax.axis_index("s")
        w = c * 16 + s
        for j in range(GROUP // 16):
            ones_v[pl.ds(j * 16, 16)] = jnp.ones((16,), jnp.float32)
        pltpu.sync_copy(zeros_hbm.at[pl.ds(s * 640, 640)],
                        deg_sh.at[pl.ds(s * 640, 640)])
        pltpu.sync_copy(dst_hbm.at[w], dst_v)
        plsc.subcore_barrier()

        def body(g, carry):
            pltpu.sync_copy(ones_v, deg_sh.at[dst_v.at[g]], add=True)
            return carry

        lax.fori_loop(0, GPT, body, 0)
        plsc.subcore_barrier()
        pltpu.sync_copy(deg_sh.at[pl.ds(s * 640, 640)],
                        out_hbm.at[pl.ds(c * DEGPAD + s * 640, 640)])

    return k(dst3, zeros1)


def _sc_aggregate(y, src3, dst3, zeros2):
    @functools.partial(
        pl.kernel,
        mesh=_mesh(),
        compiler_params=pltpu.CompilerParams(use_tc_tiling_on_sc=False),
        out_type=jax.ShapeDtypeStruct((2 * NACC, DH), jnp.float32),
        scratch_types=[
            pltpu.VMEM((GPT, GROUP), jnp.int32),
            pltpu.VMEM((GPT, GROUP), jnp.int32),
            pltpu.VMEM((GROUP, DH), jnp.float32),
            pltpu.VMEM_SHARED((NACC, DH), jnp.float32),
            pltpu.SemaphoreType.DMA,
        ],
    )
    def k(y_hbm, src_hbm, dst_hbm, zeros_hbm, out_hbm,
          src_v, dst_v, rows_v, acc_sh, sem):
        c = lax.axis_index("c")
        s = lax.axis_index("s")
        w = c * 16 + s
        pltpu.sync_copy(zeros_hbm.at[pl.ds(s * RPT, RPT)],
                        acc_sh.at[pl.ds(s * RPT, RPT)])
        pltpu.sync_copy(src_hbm.at[w], src_v)
        pltpu.sync_copy(dst_hbm.at[w], dst_v)
        plsc.subcore_barrier()

        def body(g, carry):
            pltpu.async_copy(y_hbm.at[src_v.at[g]], rows_v, sem).wait()
            pltpu.sync_copy(rows_v, acc_sh.at[dst_v.at[g]], add=True)
            return carry

        lax.fori_loop(0, GPT, body, 0)
        plsc.subcore_barrier()
        pltpu.sync_copy(acc_sh.at[pl.ds(s * RPT, RPT)],
                        out_hbm.at[pl.ds(c * NACC + s * RPT, RPT)])

    return k(y, src3, dst3, zeros2)


def _dinv(cnt_ref):
    deg = jnp.sum(cnt_ref[...], axis=1, keepdims=True) + 1.0
    row = lax.broadcasted_iota(jnp.int32, (N, 1), 0)
    deg = deg - jnp.where(row < NPAD_E, 1.0, 0.0)
    return lax.rsqrt(deg)


def _tc_pre(x, W1, cnt2):
    def body(x_ref, w_ref, cnt_ref, y_ref):
        dinv = _dinv(cnt_ref)
        xw = jnp.dot(x_ref[...], w_ref[...], preferred_element_type=jnp.float32)
        y_ref[pl.ds(0, N), :] = xw * dinv
        y_ref[pl.ds(N, NPADROWS), :] = jnp.zeros((NPADROWS, DH), jnp.float32)

    return pl.pallas_call(
        body, out_shape=jax.ShapeDtypeStruct((NY, DH), jnp.float32)
    )(x, W1, cnt2)


def _tc_mid(cnt2, y1, a0, a1, b1, W2p):
    def body(cnt_ref, y1_ref, a0_ref, a1_ref, b1_ref, w2_ref, y2_ref):
        dinv = _dinv(cnt_ref)
        out1 = (a0_ref[...] + a1_ref[...] + y1_ref[pl.ds(0, N), :]) * dinv
        h = jnp.maximum(out1 + b1_ref[...], 0.0)
        z = jnp.dot(h, w2_ref[...], preferred_element_type=jnp.float32)
        y2_ref[pl.ds(0, N), :] = z * dinv
        y2_ref[pl.ds(N, NPADROWS), :] = jnp.zeros((NPADROWS, DH), jnp.float32)

    return pl.pallas_call(
        body, out_shape=jax.ShapeDtypeStruct((NY, DH), jnp.float32)
    )(cnt2, y1, a0, a1, b1, W2p)


def _tc_post(cnt2, y2, a0, a1, b2p):
    def body(cnt_ref, y2_ref, a0_ref, a1_ref, b2_ref, o_ref):
        dinv = _dinv(cnt_ref)
        out2 = (a0_ref[...] + a1_ref[...] + y2_ref[pl.ds(0, N), :]) * dinv
        out2 = out2 + b2_ref[...]
        col = lax.broadcasted_iota(jnp.int32, (N, DH), 1)
        vals = jnp.where(col < NCLS, out2, -1e30)
        mx = jnp.max(vals, axis=1, keepdims=True)
        ex = jnp.exp(vals - mx)
        lse = mx + jnp.log(jnp.sum(ex, axis=1, keepdims=True))
        o_ref[...] = (out2 - lse)[:, 0:NCLS]

    return pl.pallas_call(
        body, out_shape=jax.ShapeDtypeStruct((N, NCLS), jnp.float32)
    )(cnt2, y2, a0, a1, b2p)


def kernel(x, edge_index, W1, b1, W2, b2):
    ei = edge_index.astype(jnp.int32)
    pad_src = (jnp.arange(NPAD_E, dtype=jnp.int32) % NPADROWS) + N
    pad_dst = jnp.arange(NPAD_E, dtype=jnp.int32) % N
    src3 = jnp.concatenate([ei[0], pad_src]).reshape(NWORK, GPT, GROUP)
    dst3 = jnp.concatenate([ei[1], pad_dst]).reshape(NWORK, GPT, GROUP)
    zeros1 = jnp.zeros((DEGPAD,), jnp.float32)
    zeros2 = jnp.zeros((NACC, DH), jnp.float32)
    W2p = jnp.zeros((DH, DH), jnp.float32).at[:, :NCLS].set(W2)
    b2p = jnp.zeros((DH,), jnp.float32).at[:NCLS].set(b2)

    cnt = _sc_degree(dst3, zeros1)
    cnt2 = cnt.reshape(2, DEGPAD)[:, :N].T

    y1 = _tc_pre(x, W1, cnt2)
    agg1 = _sc_aggregate(y1, src3, dst3, zeros2)
    y2 = _tc_mid(cnt2, y1, agg1[:N], agg1[NACC:NACC + N], b1, W2p)
    agg2 = _sc_aggregate(y2, src3, dst3, zeros2)
    return _tc_post(cnt2, y2, agg2[:N], agg2[NACC:NACC + N], b2p)


# speedup vs baseline: 46.0032x; 1.3284x over previous
import functools

import jax
import jax.numpy as jnp
from jax import lax
from jax.experimental import pallas as pl
from jax.experimental.pallas import tpu as pltpu
from jax.experimental.pallas import tpu_sc as plsc

N = 10000
E = 320000
DF = 128
DH = 16
NCLS = 7

NPADROWS = 16
NY = N + NPADROWS

NWORK = 32
GROUP = 128
GPT = 80
NBUF = 8
EPT = GPT * GROUP
EPAD = NWORK * EPT
NPAD_E = EPAD - E

DEGPAD = 10240
NACC = 10240
RPT = NACC // 16


def _mesh():
    return plsc.VectorSubcoreMesh(core_axis_name="c", subcore_axis_name="s")


def _sc_degree(dst3, zeros1):
    @functools.partial(
        pl.kernel,
        mesh=_mesh(),
        compiler_params=pltpu.CompilerParams(use_tc_tiling_on_sc=False),
        out_type=jax.ShapeDtypeStruct((2 * DEGPAD,), jnp.float32),
        scratch_types=[
            pltpu.VMEM((GPT, GROUP), jnp.int32),
            pltpu.VMEM((GROUP,), jnp.float32),
            pltpu.VMEM_SHARED((DEGPAD,), jnp.float32),
        ],
    )
    def k(dst_hbm, zeros_hbm, out_hbm, dst_v, ones_v, deg_sh):
        c = lax.axis_index("c")
        s = lax.axis_index("s")
        w = c * 16 + s
        for j in range(GROUP // 16):
            ones_v[pl.ds(j * 16, 16)] = jnp.ones((16,), jnp.float32)
        pltpu.sync_copy(zeros_hbm.at[pl.ds(s * 640, 640)],
                        deg_sh.at[pl.ds(s * 640, 640)])
        pltpu.sync_copy(dst_hbm.at[w], dst_v)
        plsc.subcore_barrier()

        def body(g, carry):
            pltpu.sync_copy(ones_v, deg_sh.at[dst_v.at[g]], add=True)
            return carry

        lax.fori_loop(0, GPT, body, 0)
        plsc.subcore_barrier()
        pltpu.sync_copy(deg_sh.at[pl.ds(s * 640, 640)],
                        out_hbm.at[pl.ds(c * DEGPAD + s * 640, 640)])

    return k(dst3, zeros1)


def _sc_aggregate(y, src3, dst3, zeros2):
    @functools.partial(
        pl.kernel,
        mesh=_mesh(),
        compiler_params=pltpu.CompilerParams(use_tc_tiling_on_sc=False),
        out_type=jax.ShapeDtypeStruct((2 * NACC, DH), jnp.float32),
        scratch_types=[
            pltpu.VMEM((GPT, GROUP), jnp.int32),
            pltpu.VMEM((GPT, GROUP), jnp.int32),
            pltpu.VMEM((NBUF, GROUP, DH), jnp.float32),
            pltpu.VMEM_SHARED((NACC, DH), jnp.float32),
            pltpu.SemaphoreType.DMA((NBUF,)),
            pltpu.SemaphoreType.DMA((NBUF,)),
        ],
    )
    def k(y_hbm, src_hbm, dst_hbm, zeros_hbm, out_hbm,
          src_v, dst_v, rows_v, acc_sh, gsem, ssem):
        c = lax.axis_index("c")
        s = lax.axis_index("s")
        w = c * 16 + s
        pltpu.sync_copy(zeros_hbm.at[pl.ds(s * RPT, RPT)],
                        acc_sh.at[pl.ds(s * RPT, RPT)])
        pltpu.sync_copy(src_hbm.at[w], src_v)
        pltpu.sync_copy(dst_hbm.at[w], dst_v)
        plsc.subcore_barrier()

        nsup = GPT // NBUF
        for b in range(NBUF):
            pltpu.async_copy(y_hbm.at[src_v.at[b]], rows_v.at[b], gsem.at[b])

        def body(step, carry):
            g0 = step * NBUF
            for b in range(NBUF):
                pltpu.make_async_copy(y_hbm.at[src_v.at[g0 + b]],
                                      rows_v.at[b], gsem.at[b]).wait()
                pltpu.async_copy(rows_v.at[b], acc_sh.at[dst_v.at[g0 + b]],
                                 ssem.at[b], add=True)
            for b in range(NBUF):
                pltpu.make_async_copy(rows_v.at[b],
                                      acc_sh.at[dst_v.at[g0 + b]],
                                      ssem.at[b]).wait()

                @pl.when(step < nsup - 1)
                def _():
                    pltpu.async_copy(y_hbm.at[src_v.at[g0 + NBUF + b]],
                                     rows_v.at[b], gsem.at[b])
            return carry

        lax.fori_loop(0, GPT // NBUF, body, 0)
        plsc.subcore_barrier()
        pltpu.sync_copy(acc_sh.at[pl.ds(s * RPT, RPT)],
                        out_hbm.at[pl.ds(c * NACC + s * RPT, RPT)])

    return k(y, src3, dst3, zeros2)


def _dinv(cnt_ref):
    deg = jnp.sum(cnt_ref[...], axis=1, keepdims=True) + 1.0
    row = lax.broadcasted_iota(jnp.int32, (N, 1), 0)
    deg = deg - jnp.where(row < NPAD_E, 1.0, 0.0)
    return lax.rsqrt(deg)


def _tc_pre(x, W1, cnt2):
    def body(x_ref, w_ref, cnt_ref, y_ref):
        dinv = _dinv(cnt_ref)
        xw = jnp.dot(x_ref[...], w_ref[...], preferred_element_type=jnp.float32)
        y_ref[pl.ds(0, N), :] = xw * dinv
        y_ref[pl.ds(N, NPADROWS), :] = jnp.zeros((NPADROWS, DH), jnp.float32)

    return pl.pallas_call(
        body, out_shape=jax.ShapeDtypeStruct((NY, DH), jnp.float32)
    )(x, W1, cnt2)


def _tc_mid(cnt2, y1, a0, a1, b1, W2p):
    def body(cnt_ref, y1_ref, a0_ref, a1_ref, b1_ref, w2_ref, y2_ref):
        dinv = _dinv(cnt_ref)
        out1 = (a0_ref[...] + a1_ref[...] + y1_ref[pl.ds(0, N), :]) * dinv
        h = jnp.maximum(out1 + b1_ref[...], 0.0)
        z = jnp.dot(h, w2_ref[...], preferred_element_type=jnp.float32)
        y2_ref[pl.ds(0, N), :] = z * dinv
        y2_ref[pl.ds(N, NPADROWS), :] = jnp.zeros((NPADROWS, DH), jnp.float32)

    return pl.pallas_call(
        body, out_shape=jax.ShapeDtypeStruct((NY, DH), jnp.float32)
    )(cnt2, y1, a0, a1, b1, W2p)


def _tc_post(cnt2, y2, a0, a1, b2p):
    def body(cnt_ref, y2_ref, a0_ref, a1_ref, b2_ref, o_ref):
        dinv = _dinv(cnt_ref)
        out2 = (a0_ref[...] + a1_ref[...] + y2_ref[pl.ds(0, N), :]) * dinv
        out2 = out2 + b2_ref[...]
        col = lax.broadcasted_iota(jnp.int32, (N, DH), 1)
        vals = jnp.where(col < NCLS, out2, -1e30)
        mx = jnp.max(vals, axis=1, keepdims=True)
        ex = jnp.exp(vals - mx)
        lse = mx + jnp.log(jnp.sum(ex, axis=1, keepdims=True))
        o_ref[...] = (out2 - lse)[:, 0:NCLS]

    return pl.pallas_call(
        body, out_shape=jax.ShapeDtypeStruct((N, NCLS), jnp.float32)
    )(cnt2, y2, a0, a1, b2p)


def kernel(x, edge_index, W1, b1, W2, b2):
    ei = edge_index.astype(jnp.int32)
    pad_src = (jnp.arange(NPAD_E, dtype=jnp.int32) % NPADROWS) + N
    pad_dst = jnp.arange(NPAD_E, dtype=jnp.int32) % N
    src3 = jnp.concatenate([ei[0], pad_src]).reshape(NWORK, GPT, GROUP)
    dst3 = jnp.concatenate([ei[1], pad_dst]).reshape(NWORK, GPT, GROUP)
    zeros1 = jnp.zeros((DEGPAD,), jnp.float32)
    zeros2 = jnp.zeros((NACC, DH), jnp.float32)
    W2p = jnp.zeros((DH, DH), jnp.float32).at[:, :NCLS].set(W2)
    b2p = jnp.zeros((DH,), jnp.float32).at[:NCLS].set(b2)

    cnt = _sc_degree(dst3, zeros1)
    cnt2 = cnt.reshape(2, DEGPAD)[:, :N].T

    y1 = _tc_pre(x, W1, cnt2)
    agg1 = _sc_aggregate(y1, src3, dst3, zeros2)
    y2 = _tc_mid(cnt2, y1, agg1[:N], agg1[NACC:NACC + N], b1, W2p)
    agg2 = _sc_aggregate(y2, src3, dst3, zeros2)
    return _tc_post(cnt2, y2, agg2[:N], agg2[NACC:NACC + N], b2p)


# speedup vs baseline: 53.8315x; 1.1702x over previous
import functools

import jax
import jax.numpy as jnp
from jax import lax
from jax.experimental import pallas as pl
from jax.experimental.pallas import tpu as pltpu
from jax.experimental.pallas import tpu_sc as plsc

N = 10000
E = 320000
DF = 128
DH = 16
NCLS = 7

NPADROWS = 7680
NY = N + NPADROWS

NWORK = 32
GROUP = 128
GPT = 80
NBUF = 8
EPT = GPT * GROUP
EPAD = NWORK * EPT
NPAD_E = EPAD - E

DEGPAD = 10240
NACC = 10240
RPT = NACC // 16


def _mesh():
    return plsc.VectorSubcoreMesh(core_axis_name="c", subcore_axis_name="s")


def _sc_degree(dst3, zeros1):
    @functools.partial(
        pl.kernel,
        mesh=_mesh(),
        compiler_params=pltpu.CompilerParams(use_tc_tiling_on_sc=False),
        out_type=jax.ShapeDtypeStruct((2 * DEGPAD,), jnp.float32),
        scratch_types=[
            pltpu.VMEM((GPT, GROUP), jnp.int32),
            pltpu.VMEM((GROUP,), jnp.float32),
            pltpu.VMEM_SHARED((DEGPAD,), jnp.float32),
        ],
    )
    def k(dst_hbm, zeros_hbm, out_hbm, dst_v, ones_v, deg_sh):
        c = lax.axis_index("c")
        s = lax.axis_index("s")
        w = c * 16 + s
        for j in range(GROUP // 16):
            ones_v[pl.ds(j * 16, 16)] = jnp.ones((16,), jnp.float32)
        pltpu.sync_copy(zeros_hbm.at[pl.ds(s * 640, 640)],
                        deg_sh.at[pl.ds(s * 640, 640)])
        pltpu.sync_copy(dst_hbm.at[w], dst_v)
        plsc.subcore_barrier()

        def body(g, carry):
            pltpu.sync_copy(ones_v, deg_sh.at[dst_v.at[g]], add=True)
            return carry

        lax.fori_loop(0, GPT, body, 0)
        plsc.subcore_barrier()
        pltpu.sync_copy(deg_sh.at[pl.ds(s * 640, 640)],
                        out_hbm.at[pl.ds(c * DEGPAD + s * 640, 640)])

    return k(dst3, zeros1)


def _sc_aggregate(y, src3, dst3, zeros2):
    @functools.partial(
        pl.kernel,
        mesh=_mesh(),
        compiler_params=pltpu.CompilerParams(use_tc_tiling_on_sc=False),
        out_type=jax.ShapeDtypeStruct((2 * NACC, DH), jnp.float32),
        scratch_types=[
            pltpu.VMEM((GPT, GROUP), jnp.int32),
            pltpu.VMEM((GPT, GROUP), jnp.int32),
            pltpu.VMEM((NBUF, GROUP, DH), jnp.float32),
            pltpu.VMEM_SHARED((NACC, DH), jnp.float32),
            pltpu.SemaphoreType.DMA((NBUF,)),
            pltpu.SemaphoreType.DMA((NBUF,)),
        ],
    )
    def k(y_hbm, src_hbm, dst_hbm, zeros_hbm, out_hbm,
          src_v, dst_v, rows_v, acc_sh, gsem, ssem):
        c = lax.axis_index("c")
        s = lax.axis_index("s")
        w = c * 16 + s
        pltpu.sync_copy(zeros_hbm.at[pl.ds(s * RPT, RPT)],
                        acc_sh.at[pl.ds(s * RPT, RPT)])
        pltpu.sync_copy(src_hbm.at[w], src_v)
        pltpu.sync_copy(dst_hbm.at[w], dst_v)
        plsc.subcore_barrier()

        nsup = GPT // NBUF
        for b in range(NBUF):
            pltpu.async_copy(y_hbm.at[src_v.at[b]], rows_v.at[b], gsem.at[b])

        def body(step, carry):
            g0 = step * NBUF
            for b in range(NBUF):
                pltpu.make_async_copy(y_hbm.at[src_v.at[g0 + b]],
                                      rows_v.at[b], gsem.at[b]).wait()
                pltpu.async_copy(rows_v.at[b], acc_sh.at[dst_v.at[g0 + b]],
                                 ssem.at[b], add=True)
            for b in range(NBUF):
                pltpu.make_async_copy(rows_v.at[b],
                                      acc_sh.at[dst_v.at[g0 + b]],
                                      ssem.at[b]).wait()

                @pl.when(step < nsup - 1)
                def _():
                    pltpu.async_copy(y_hbm.at[src_v.at[g0 + NBUF + b]],
                                     rows_v.at[b], gsem.at[b])
            return carry

        lax.fori_loop(0, GPT // NBUF, body, 0)
        plsc.subcore_barrier()
        pltpu.sync_copy(acc_sh.at[pl.ds(s * RPT, RPT)],
                        out_hbm.at[pl.ds(c * NACC + s * RPT, RPT)])

    return k(y, src3, dst3, zeros2)


def _dinv(cnt_ref):
    deg = jnp.sum(cnt_ref[...], axis=1, keepdims=True) + 1.0
    row = lax.broadcasted_iota(jnp.int32, (N, 1), 0)
    deg = deg - jnp.where(row < NPAD_E, 1.0, 0.0)
    return lax.rsqrt(deg)


def _tc_pre(x, W1, cnt2):
    def body(x_ref, w_ref, cnt_ref, y_ref):
        dinv = _dinv(cnt_ref)
        xw = jnp.dot(x_ref[...], w_ref[...], preferred_element_type=jnp.float32)
        y_ref[pl.ds(0, N), :] = xw * dinv
        y_ref[pl.ds(N, NPADROWS), :] = jnp.zeros((NPADROWS, DH), jnp.float32)

    return pl.pallas_call(
        body, out_shape=jax.ShapeDtypeStruct((NY, DH), jnp.float32)
    )(x, W1, cnt2)


def _tc_mid(cnt2, y1, a0, a1, b1, W2p):
    def body(cnt_ref, y1_ref, a0_ref, a1_ref, b1_ref, w2_ref, y2_ref):
        dinv = _dinv(cnt_ref)
        out1 = (a0_ref[...] + a1_ref[...] + y1_ref[pl.ds(0, N), :]) * dinv
        h = jnp.maximum(out1 + b1_ref[...], 0.0)
        z = jnp.dot(h, w2_ref[...], preferred_element_type=jnp.float32)
        y2_ref[pl.ds(0, N), :] = z * dinv
        y2_ref[pl.ds(N, NPADROWS), :] = jnp.zeros((NPADROWS, DH), jnp.float32)

    return pl.pallas_call(
        body, out_shape=jax.ShapeDtypeStruct((NY, DH), jnp.float32)
    )(cnt2, y1, a0, a1, b1, W2p)


def _tc_post(cnt2, y2, a0, a1, b2p):
    def body(cnt_ref, y2_ref, a0_ref, a1_ref, b2_ref, o_ref):
        dinv = _dinv(cnt_ref)
        out2 = (a0_ref[...] + a1_ref[...] + y2_ref[pl.ds(0, N), :]) * dinv
        out2 = out2 + b2_ref[...]
        col = lax.broadcasted_iota(jnp.int32, (N, DH), 1)
        vals = jnp.where(col < NCLS, out2, -1e30)
        mx = jnp.max(vals, axis=1, keepdims=True)
        ex = jnp.exp(vals - mx)
        lse = mx + jnp.log(jnp.sum(ex, axis=1, keepdims=True))
        o_ref[...] = (out2 - lse)[:, 0:NCLS]

    return pl.pallas_call(
        body, out_shape=jax.ShapeDtypeStruct((N, NCLS), jnp.float32)
    )(cnt2, y2, a0, a1, b2p)


def kernel(x, edge_index, W1, b1, W2, b2):
    ei = edge_index.astype(jnp.int32)
    pad_src = jnp.arange(NPAD_E, dtype=jnp.int32) + N
    pad_dst = jnp.arange(NPAD_E, dtype=jnp.int32) % N
    src3 = jnp.concatenate([ei[0], pad_src]).reshape(NWORK, GPT, GROUP)
    dst3 = jnp.concatenate([ei[1], pad_dst]).reshape(NWORK, GPT, GROUP)
    zeros1 = jnp.zeros((DEGPAD,), jnp.float32)
    zeros2 = jnp.zeros((NACC, DH), jnp.float32)
    W2p = jnp.zeros((DH, DH), jnp.float32).at[:, :NCLS].set(W2)
    b2p = jnp.zeros((DH,), jnp.float32).at[:NCLS].set(b2)

    cnt = _sc_degree(dst3, zeros1)
    cnt2 = cnt.reshape(2, DEGPAD)[:, :N].T

    y1 = _tc_pre(x, W1, cnt2)
    agg1 = _sc_aggregate(y1, src3, dst3, zeros2)
    y2 = _tc_mid(cnt2, y1, agg1[:N], agg1[NACC:NACC + N], b1, W2p)
    agg2 = _sc_aggregate(y2, src3, dst3, zeros2)
    return _tc_post(cnt2, y2, agg2[:N], agg2[NACC:NACC + N], b2p)
